# Initial kernel scaffold; baseline (speedup 1.0000x reference)
import jax
import jax.numpy as jnp
from jax.experimental import pallas as pl


def kernel(x, offset_x1, offset_x2, offset_y1, offset_y2, thresholds, radii):
    raise NotImplementedError("write your pallas kernel here")


# speedup vs baseline: 1112.3720x; 1112.3720x over previous
import functools

import jax
import jax.numpy as jnp
from jax.experimental import pallas as pl
from jax.experimental.pallas import tpu as pltpu

_H = 224
_W = 224
_PAD = 16
_MAXR = 4
_HP = _H + 2 * _PAD
_WP = _W + 2 * _PAD


def _body(x_ref, s1y_ref, s1x_ref, s2y_ref, s2x_ref, ridx_ref, thr_ref,
          out_ref, m_s):
    p = pl.program_id(1)

    @pl.when(p == 0)
    def _compute_box_means():
        img = x_ref[0, 0]
        top = img[0:1, :]
        bot = img[_H - 1:_H, :]
        pimg = jnp.concatenate([top] * _MAXR + [img] + [bot] * _MAXR, axis=0)
        left = pimg[:, 0:1]
        right = pimg[:, _W - 1:_W]
        pimg = jnp.concatenate([left] * _MAXR + [pimg] + [right] * _MAXR,
                               axis=1)
        for r in range(1, _MAXR + 1):
            rs = pimg[_MAXR - r:_MAXR - r + _H, :]
            for k in range(-r + 1, r + 1):
                rs = rs + pimg[_MAXR + k:_MAXR + k + _H, :]
            bs = rs[:, _MAXR - r:_MAXR - r + _W]
            for k in range(-r + 1, r + 1):
                bs = bs + rs[:, _MAXR + k:_MAXR + k + _W]
            m = bs * (1.0 / float((2 * r + 1) ** 2))
            rowpad = jnp.concatenate(
                [jnp.broadcast_to(m[0:1, :], (_PAD, _W)), m,
                 jnp.broadcast_to(m[_H - 1:_H, :], (_PAD, _W))], axis=0)
            m_s[r - 1, :, _PAD:_PAD + _W] = rowpad
            lcol = m_s[r - 1, :, _PAD:_PAD + 1]
            m_s[r - 1, :, 0:_PAD] = jnp.broadcast_to(lcol, (_HP, _PAD))
            rcol = m_s[r - 1, :, _PAD + _W - 1:_PAD + _W]
            m_s[r - 1, :, _PAD + _W:_WP] = jnp.broadcast_to(rcol, (_HP, _PAD))

    r_i = ridx_ref[p]
    plane = m_s[r_i]

    def _sample(sy, sx):
        rolled = pltpu.roll(plane, _HP - sy, 0)
        rolled = pltpu.roll(rolled, _WP - sx, 1)
        return rolled[0:_H, 0:_W]

    a = _sample(s1y_ref[p], s1x_ref[p])
    c = _sample(s2y_ref[p], s2x_ref[p])
    out_ref[0, 0] = a - c - thr_ref[p]


@jax.jit
def kernel(x, offset_x1, offset_x2, offset_y1, offset_y2, thresholds, radii):
    B = x.shape[0]
    num_pairs = radii.shape[0]
    s1y = (jnp.floor(offset_y1) + _PAD).astype(jnp.int32)
    s1x = (jnp.floor(offset_x1) + _PAD).astype(jnp.int32)
    s2y = (jnp.floor(offset_y2) + _PAD).astype(jnp.int32)
    s2x = (jnp.floor(offset_x2) + _PAD).astype(jnp.int32)
    ridx = (radii - 1).astype(jnp.int32)
    thr = thresholds.astype(jnp.float32)

    grid = (B, num_pairs)
    smem = pl.BlockSpec(memory_space=pltpu.SMEM)
    return pl.pallas_call(
        _body,
        grid=grid,
        in_specs=[
            pl.BlockSpec((1, 1, _H, _W), lambda b, p: (b, 0, 0, 0)),
            smem, smem, smem, smem, smem, smem,
        ],
        out_specs=pl.BlockSpec((1, 1, _H, _W), lambda b, p: (b, p, 0, 0)),
        out_shape=jax.ShapeDtypeStruct((B, num_pairs, _H, _W), jnp.float32),
        scratch_shapes=[pltpu.VMEM((_MAXR, _HP, _WP), jnp.float32)],
        compiler_params=pltpu.CompilerParams(
            dimension_semantics=("arbitrary", "arbitrary")),
    )(x, s1y, s1x, s2y, s2x, ridx, thr)
